# Initial kernel scaffold; baseline (speedup 1.0000x reference)
import jax
import jax.numpy as jnp
from jax.experimental import pallas as pl


def kernel(inputs, embeddings):
    raise NotImplementedError("write your pallas kernel here")


# speedup vs baseline: 7.9578x; 7.9578x over previous
import functools

import jax
import jax.numpy as jnp
from jax import lax
from jax.experimental import pallas as pl
from jax.experimental.pallas import tpu as pltpu
from jax.experimental.pallas import tpu_sc as plsc

MAX_SPAN = 255
Q = 2048
K = 2048
DEPTH = 64
EXT = 4096
NUM_WORKERS = 32
ROWS_PER_W = EXT // NUM_WORKERS
GATHER_W = 128
BQ = 8


def _build_ext_sc(embeddings_padded, idx):
    mesh = plsc.VectorSubcoreMesh(core_axis_name="c", subcore_axis_name="s")

    @functools.partial(
        pl.kernel,
        mesh=mesh,
        out_type=jax.ShapeDtypeStruct((EXT, GATHER_W), jnp.float32),
        scratch_types=[
            pltpu.VMEM((ROWS_PER_W,), jnp.int32),
            pltpu.VMEM((ROWS_PER_W, GATHER_W), jnp.float32),
            pltpu.SemaphoreType.DMA,
        ],
    )
    def gather_kernel(table_hbm, idx_hbm, ext_hbm, idx_v, rows_v, sem):
        wid = lax.axis_index("s") * 2 + lax.axis_index("c")
        base = wid * ROWS_PER_W
        pltpu.sync_copy(idx_hbm.at[pl.ds(base, ROWS_PER_W)], idx_v)
        pltpu.async_copy(table_hbm.at[idx_v], rows_v, sem).wait()
        pltpu.sync_copy(rows_v, ext_hbm.at[pl.ds(base, ROWS_PER_W)])

    return gather_kernel(embeddings_padded, idx)


def _fanout_body(ext_ref, out_ref):
    q0 = pl.program_id(0) * BQ
    for i in range(BQ):
        out_ref[i] = ext_ref[pl.ds(K - 1 - (q0 + i), K), :DEPTH]


def kernel(inputs, embeddings):
    del inputs
    idx = jnp.clip(jnp.arange(EXT, dtype=jnp.int32) - (K - 1) + (MAX_SPAN - 1),
                   0, 2 * MAX_SPAN - 2)
    emb_padded = jnp.pad(embeddings, ((0, 0), (0, GATHER_W - DEPTH)))
    ext = _build_ext_sc(emb_padded, idx)
    out = pl.pallas_call(
        _fanout_body,
        grid=(Q // BQ,),
        in_specs=[pl.BlockSpec((EXT, GATHER_W), lambda q: (0, 0))],
        out_specs=pl.BlockSpec((BQ, K, DEPTH), lambda q: (q, 0, 0)),
        out_shape=jax.ShapeDtypeStruct((Q, K, DEPTH), jnp.float32),
    )(ext)
    return out


# speedup vs baseline: 7.9709x; 1.0016x over previous
import functools

import jax
import jax.numpy as jnp
from jax import lax
from jax.experimental import pallas as pl
from jax.experimental.pallas import tpu as pltpu
from jax.experimental.pallas import tpu_sc as plsc

MAX_SPAN = 255
Q = 2048
K = 2048
DEPTH = 64
EXT = 4096
NUM_WORKERS = 32
ROWS_PER_W = EXT // NUM_WORKERS
GATHER_W = 128
BQ = 16


def _build_ext_sc(embeddings_padded, idx):
    mesh = plsc.VectorSubcoreMesh(core_axis_name="c", subcore_axis_name="s")

    @functools.partial(
        pl.kernel,
        mesh=mesh,
        out_type=jax.ShapeDtypeStruct((EXT, GATHER_W), jnp.float32),
        scratch_types=[
            pltpu.VMEM((ROWS_PER_W,), jnp.int32),
            pltpu.VMEM((ROWS_PER_W, GATHER_W), jnp.float32),
            pltpu.SemaphoreType.DMA,
        ],
    )
    def gather_kernel(table_hbm, idx_hbm, ext_hbm, idx_v, rows_v, sem):
        wid = lax.axis_index("s") * 2 + lax.axis_index("c")
        base = wid * ROWS_PER_W
        pltpu.sync_copy(idx_hbm.at[pl.ds(base, ROWS_PER_W)], idx_v)
        pltpu.async_copy(table_hbm.at[idx_v], rows_v, sem).wait()
        pltpu.sync_copy(rows_v, ext_hbm.at[pl.ds(base, ROWS_PER_W)])

    return gather_kernel(embeddings_padded, idx)


def _fanout_body(ext_ref, out_ref):
    q0 = pl.program_id(0) * BQ
    for i in range(BQ):
        out_ref[i] = ext_ref[pl.ds(K - 1 - (q0 + i), K), :DEPTH]


def kernel(inputs, embeddings):
    del inputs
    idx = jnp.clip(jnp.arange(EXT, dtype=jnp.int32) - (K - 1) + (MAX_SPAN - 1),
                   0, 2 * MAX_SPAN - 2)
    emb_padded = jnp.pad(embeddings, ((0, 0), (0, GATHER_W - DEPTH)))
    ext = _build_ext_sc(emb_padded, idx)
    out = pl.pallas_call(
        _fanout_body,
        grid=(Q // BQ,),
        in_specs=[pl.BlockSpec((EXT, GATHER_W), lambda q: (0, 0))],
        out_specs=pl.BlockSpec((BQ, K, DEPTH), lambda q: (q, 0, 0)),
        out_shape=jax.ShapeDtypeStruct((Q, K, DEPTH), jnp.float32),
    )(ext)
    return out
